# Initial kernel scaffold; baseline (speedup 1.0000x reference)
import jax
import jax.numpy as jnp
from jax.experimental import pallas as pl


def kernel(s_t, strategy0, Wm, Ws, Wp_h, Wp_a, A_h, A_a, We_h, Wa_h, We_a, Wa_a, Wv, w_h):
    raise NotImplementedError("write your pallas kernel here")


# speedup vs baseline: 4.7175x; 4.7175x over previous
import jax
import jax.numpy as jnp
from jax.experimental import pallas as pl

F32 = jnp.float32
SD = 1024
GD = 512
AD = 128
PD = 4096
TRAJ = 256
CAND = 2048


def _dot(a, b):
    return jax.lax.dot_general(a, b, (((1,), (0,)), ((), ())),
                               preferred_element_type=F32)


def _prologue_krn(s_t, strat0, Wm, Ws, Wph_t, Wph_b, Wpa_b, We_h, wh_row,
                  strat_o, bias_h_o, bias_a_o, logits0_o, base0_o, hs_o):
    strat_o[...] = jnp.tanh(_dot(strat0[...], Wm[...]) + _dot(s_t[...], Ws[...]))
    strat = strat_o[...]
    bias_h_o[...] = _dot(strat, Wph_b[...])
    bias_a_o[...] = _dot(strat, Wpa_b[...])
    logits0_o[...] = _dot(s_t[...], Wph_t[...]) + bias_h_o[...]
    base0_o[...] = _dot(s_t[...], We_h[...])
    hs_o[...] = jnp.sum(s_t[...] * wh_row[...]).reshape(1, 1)


def _prologue(s_t, strat0, Wm, Ws, Wph_t, Wph_b, Wpa_b, We_h, wh_row):
    return pl.pallas_call(
        _prologue_krn,
        out_shape=[
            jax.ShapeDtypeStruct((1, GD), F32),
            jax.ShapeDtypeStruct((1, PD), F32),
            jax.ShapeDtypeStruct((1, PD), F32),
            jax.ShapeDtypeStruct((1, PD), F32),
            jax.ShapeDtypeStruct((1, SD), F32),
            jax.ShapeDtypeStruct((1, 1), F32),
        ],
    )(s_t, strat0, Wm, Ws, Wph_t, Wph_b, Wpa_b, We_h, wh_row)


def _a0_krn(lrow, lcol, A_h, acts_o, r0_o):
    v = lrow[...]
    ii = jax.lax.broadcasted_iota(jnp.int32, (1, PD), 1)
    rank = jnp.zeros((1, PD), jnp.int32)
    for t in range(PD // 256):
        vj = lcol[pl.ds(t * 256, 256), :]
        ji = jax.lax.broadcasted_iota(jnp.int32, (256, 1), 0) + t * 256
        hit = (vj > v) | ((vj == v) & (ji < ii))
        rank = rank + jnp.sum(hit.astype(jnp.int32), axis=0, keepdims=True)
    iif = ii.astype(F32)
    for t in range(CAND // 256):
        p = jax.lax.broadcasted_iota(jnp.int32, (256, 1), 0) + (t * 256)
        oh = (rank == p).astype(F32)
        acts_o[pl.ds(t * 256, 256), :] = _dot(oh, A_h[...])
        r0_o[pl.ds(t * 256, 256), :] = jnp.sum(oh * iif, axis=1, keepdims=True)


def _a0(lrow, lcol, A_h):
    return pl.pallas_call(
        _a0_krn,
        out_shape=[
            jax.ShapeDtypeStruct((CAND, AD), F32),
            jax.ShapeDtypeStruct((CAND, 1), F32),
        ],
    )(lrow, lcol, A_h)


def _expand_krn(rem, acts, base, Wa_h, Wpa_t, bias_a, A_a, We_a, Wa_a,
                strat, wh_col, Wv, hs, nxt_o, vals_o):
    partial = jnp.tanh(base[...] + _dot(acts[...], Wa_h[...]))
    logits = _dot(partial, Wpa_t[...]) + bias_a[...]
    m = jnp.max(logits, axis=1, keepdims=True)
    ii = jax.lax.broadcasted_iota(jnp.int32, (256, PD), 1)
    idx = jnp.min(jnp.where(logits == m, ii, PD), axis=1, keepdims=True)
    oh = (ii == idx).astype(F32)
    adv = _dot(oh, A_a[...])
    nxt_o[...] = jnp.tanh(_dot(partial, We_a[...]) + _dot(adv, Wa_a[...]))
    nxt = nxt_o[...]
    va = _dot(nxt, wh_col[...])
    hcat = jnp.concatenate(
        [nxt, jnp.broadcast_to(strat[...], (nxt.shape[0], GD))], axis=1)
    vp = _dot(hcat, Wv[...])
    vals_o[...] = (va - hs[...]) + vp * rem


def _expand(rem, acts, base, Wa_h, Wpa_t, bias_a, A_a, We_a, Wa_a,
            strat, wh_col, Wv, hs):
    import functools
    blk = 256
    nblk = CAND // blk
    const = lambda shape: pl.BlockSpec(shape, lambda i: (0, 0))
    return pl.pallas_call(
        functools.partial(_expand_krn, float(rem)),
        grid=(nblk,),
        in_specs=[
            pl.BlockSpec((blk, AD), lambda i: (i, 0)),
            const((TRAJ, SD)),
            const((AD, SD)),
            const((SD, PD)),
            const((1, PD)),
            const((PD, AD)),
            const((SD, SD)),
            const((AD, SD)),
            const((1, GD)),
            const((SD, 1)),
            const((SD + GD, 1)),
            const((1, 1)),
        ],
        out_specs=[
            pl.BlockSpec((blk, SD), lambda i: (i, 0)),
            pl.BlockSpec((blk, 1), lambda i: (i, 0)),
        ],
        out_shape=[
            jax.ShapeDtypeStruct((CAND, SD), F32),
            jax.ShapeDtypeStruct((CAND, 1), F32),
        ],
    )(acts, base, Wa_h, Wpa_t, bias_a, A_a, We_a, Wa_a, strat, wh_col, Wv, hs)


def _select_krn(vrow, vcol, nxt, We_h, cand_o, sel_o, base_o):
    v = vrow[...]
    ii = jax.lax.broadcasted_iota(jnp.int32, (1, CAND), 1)
    rank = jnp.zeros((1, CAND), jnp.int32)
    for t in range(CAND // 256):
        vj = vcol[pl.ds(t * 256, 256), :]
        ji = jax.lax.broadcasted_iota(jnp.int32, (256, 1), 0) + t * 256
        hit = (vj > v) | ((vj == v) & (ji < ii))
        rank = rank + jnp.sum(hit.astype(jnp.int32), axis=0, keepdims=True)
    p = jax.lax.broadcasted_iota(jnp.int32, (TRAJ, 1), 0)
    oh = (rank == p).astype(F32)
    cand_o[...] = _dot(oh, nxt[...])
    sel_o[...] = jnp.sum(oh * ii.astype(F32), axis=1, keepdims=True)
    base_o[...] = _dot(cand_o[...], We_h[...])


def _select(vrow, vcol, nxt, We_h):
    return pl.pallas_call(
        _select_krn,
        out_shape=[
            jax.ShapeDtypeStruct((TRAJ, SD), F32),
            jax.ShapeDtypeStruct((TRAJ, 1), F32),
            jax.ShapeDtypeStruct((TRAJ, SD), F32),
        ],
    )(vrow, vcol, nxt, We_h)


def _head_krn(cand, Wph_t, bias_h, A_h, acts_o):
    logits = _dot(cand[...], Wph_t[...]) + bias_h[...]
    ii = jax.lax.broadcasted_iota(jnp.int32, (TRAJ, PD), 1)
    neg = jnp.float32(-jnp.inf)
    for j in range(8):
        m = jnp.max(logits, axis=1, keepdims=True)
        idx = jnp.min(jnp.where(logits == m, ii, PD), axis=1, keepdims=True)
        sel = ii == idx
        oh = sel.astype(F32)
        acts_o[pl.ds(j * TRAJ, TRAJ), :] = _dot(oh, A_h[...])
        logits = jnp.where(sel, neg, logits)


def _head(cand, Wph_t, bias_h, A_h):
    return pl.pallas_call(
        _head_krn,
        out_shape=jax.ShapeDtypeStruct((CAND, AD), F32),
    )(cand, Wph_t, bias_h, A_h)


def _trace_krn(vrow, sel1, sel0, r0, A_h, out_o):
    v = vrow[...]
    ii = jax.lax.broadcasted_iota(jnp.int32, (1, CAND), 1)
    m = jnp.max(v)
    c2 = jnp.min(jnp.where(v == m, ii, CAND))
    q2 = jnp.remainder(c2, TRAJ)
    i256 = jax.lax.broadcasted_iota(jnp.int32, (1, TRAJ), 1)
    g1 = jnp.sum(jnp.where(i256 == q2, sel1[...], 0.0)).astype(jnp.int32)
    p = jnp.remainder(g1, TRAJ)
    g0 = jnp.sum(jnp.where(i256 == p, sel0[...], 0.0)).astype(jnp.int32)
    ai = jnp.sum(jnp.where(ii == g0, r0[...], 0.0)).astype(jnp.int32)
    ipd = jax.lax.broadcasted_iota(jnp.int32, (1, PD), 1)
    oh = (ipd == ai).astype(F32)
    out_o[...] = jax.lax.dot_general(
        oh, A_h[...], (((1,), (0,)), ((), ())), preferred_element_type=F32,
        precision=jax.lax.Precision.HIGHEST)


def _trace(vrow, sel1, sel0, r0, A_h):
    return pl.pallas_call(
        _trace_krn,
        out_shape=jax.ShapeDtypeStruct((1, AD), F32),
    )(vrow, sel1, sel0, r0, A_h)


def kernel(s_t, strategy0, Wm, Ws, Wp_h, Wp_a, A_h, A_a, We_h, Wa_h, We_a,
           Wa_a, Wv, w_h):
    s_t = s_t.reshape(1, SD)
    Wph_t, Wph_b = Wp_h[:SD], Wp_h[SD:]
    Wpa_t, Wpa_b = Wp_a[:SD], Wp_a[SD:]
    wh_row = w_h.reshape(1, SD)
    wh_col = w_h.reshape(SD, 1)

    strat, bias_h, bias_a, logits0, base0, hs = _prologue(
        s_t, strategy0, Wm, Ws, Wph_t, Wph_b, Wpa_b, We_h, wh_row)

    acts0, r0 = _a0(logits0, logits0.reshape(PD, 1), A_h)
    base0r = jnp.broadcast_to(base0, (TRAJ, SD))
    nxt0, vals0 = _expand(7, acts0, base0r, Wa_h, Wpa_t, bias_a, A_a, We_a,
                          Wa_a, strat, wh_col, Wv, hs)

    cand1, sel0, base1 = _select(vals0.reshape(1, CAND), vals0, nxt0, We_h)
    acts1 = _head(cand1, Wph_t, bias_h, A_h)
    nxt1, vals1 = _expand(6, acts1, base1, Wa_h, Wpa_t, bias_a, A_a, We_a,
                          Wa_a, strat, wh_col, Wv, hs)

    cand2, sel1, base2 = _select(vals1.reshape(1, CAND), vals1, nxt1, We_h)
    acts2 = _head(cand2, Wph_t, bias_h, A_h)
    _, vals2 = _expand(5, acts2, base2, Wa_h, Wpa_t, bias_a, A_a, We_a,
                       Wa_a, strat, wh_col, Wv, hs)

    out = _trace(vals2.reshape(1, CAND), sel1.reshape(1, TRAJ),
                 sel0.reshape(1, TRAJ), r0.reshape(1, CAND), A_h)
    return out.reshape(AD)


# speedup vs baseline: 5.1487x; 1.0914x over previous
import functools

import jax
import jax.numpy as jnp
from jax.experimental import pallas as pl

F32 = jnp.float32
BF16 = jnp.bfloat16
SD = 1024
GD = 512
AD = 128
PD = 4096
TRAJ = 256
CAND = 2048


def _dot(a, b):
    return jax.lax.dot_general(a.astype(BF16), b, (((1,), (0,)), ((), ())),
                               preferred_element_type=F32)


def _prologue_krn(s_t, strat0, Wm, Ws, Wph_t, Wph_b, Wpa_b, We_h, wh_row,
                  strat_o, bias_h_o, bias_a_o, logits0_o, base0_o, hs_o):
    strat_o[...] = jnp.tanh(_dot(strat0[...], Wm[...]) + _dot(s_t[...], Ws[...]))
    strat = strat_o[...]
    bias_h_o[...] = _dot(strat, Wph_b[...])
    bias_a_o[...] = _dot(strat, Wpa_b[...])
    logits0_o[...] = _dot(s_t[...], Wph_t[...]) + bias_h_o[...]
    base0_o[...] = _dot(s_t[...], We_h[...])
    hs_o[...] = jnp.sum(s_t[...] * wh_row[...]).reshape(1, 1)


def _prologue(s_t, strat0, Wm, Ws, Wph_t, Wph_b, Wpa_b, We_h, wh_row):
    return pl.pallas_call(
        _prologue_krn,
        out_shape=[
            jax.ShapeDtypeStruct((1, GD), F32),
            jax.ShapeDtypeStruct((1, PD), F32),
            jax.ShapeDtypeStruct((1, PD), F32),
            jax.ShapeDtypeStruct((1, PD), F32),
            jax.ShapeDtypeStruct((1, SD), F32),
            jax.ShapeDtypeStruct((1, 1), F32),
        ],
    )(s_t, strat0, Wm, Ws, Wph_t, Wph_b, Wpa_b, We_h, wh_row)


def _a0_krn(lrow, lcol, A_h, acts_o, r0_o):
    v = lrow[...]
    ii = jax.lax.broadcasted_iota(jnp.int32, (1, PD), 1)
    rank = jnp.zeros((1, PD), jnp.int32)
    for t in range(PD // 256):
        vj = lcol[pl.ds(t * 256, 256), :]
        ji = jax.lax.broadcasted_iota(jnp.int32, (256, 1), 0) + t * 256
        hit = (vj > v) | ((vj == v) & (ji < ii))
        rank = rank + jnp.sum(hit.astype(jnp.int32), axis=0, keepdims=True)
    iif = ii.astype(F32)
    for t in range(CAND // 256):
        p = jax.lax.broadcasted_iota(jnp.int32, (256, 1), 0) + (t * 256)
        oh = (rank == p).astype(F32)
        acts_o[pl.ds(t * 256, 256), :] = _dot(oh, A_h[...]).astype(BF16)
        r0_o[pl.ds(t * 256, 256), :] = jnp.sum(oh * iif, axis=1, keepdims=True)


def _a0(lrow, lcol, A_h):
    return pl.pallas_call(
        _a0_krn,
        out_shape=[
            jax.ShapeDtypeStruct((CAND, AD), BF16),
            jax.ShapeDtypeStruct((CAND, 1), F32),
        ],
    )(lrow, lcol, A_h)


def _expand_krn(rem, acts, base, Wa_h, Wpa_t, bias_a, A_a, We_a, Wa_a,
                strat, wh_col, Wv, hs, nxt_o, vals_o):
    partial = jnp.tanh(base[...] + _dot(acts[...], Wa_h[...]))
    logits = _dot(partial, Wpa_t[...]) + bias_a[...]
    m = jnp.max(logits, axis=1, keepdims=True)
    ii = jax.lax.broadcasted_iota(jnp.int32, (256, PD), 1)
    idx = jnp.min(jnp.where(logits == m, ii, PD), axis=1, keepdims=True)
    oh = (ii == idx).astype(F32)
    adv = _dot(oh, A_a[...])
    nxt_o[...] = jnp.tanh(_dot(partial, We_a[...])
                          + _dot(adv, Wa_a[...])).astype(BF16)
    nxt = nxt_o[...]
    va = _dot(nxt, wh_col[...])
    hcat = jnp.concatenate(
        [nxt, jnp.broadcast_to(strat[...], (nxt.shape[0], GD))], axis=1)
    vp = _dot(hcat, Wv[...])
    vals_o[...] = (va - hs[...]) + vp * rem


def _expand(rem, acts, base, Wa_h, Wpa_t, bias_a, A_a, We_a, Wa_a,
            strat, wh_col, Wv, hs):
    blk = 256
    nblk = CAND // blk
    const = lambda shape: pl.BlockSpec(shape, lambda i: (0, 0))
    return pl.pallas_call(
        functools.partial(_expand_krn, float(rem)),
        grid=(nblk,),
        in_specs=[
            pl.BlockSpec((blk, AD), lambda i: (i, 0)),
            const((TRAJ, SD)),
            const((AD, SD)),
            const((SD, PD)),
            const((1, PD)),
            const((PD, AD)),
            const((SD, SD)),
            const((AD, SD)),
            const((1, GD)),
            const((SD, 1)),
            const((SD + GD, 1)),
            const((1, 1)),
        ],
        out_specs=[
            pl.BlockSpec((blk, SD), lambda i: (i, 0)),
            pl.BlockSpec((blk, 1), lambda i: (i, 0)),
        ],
        out_shape=[
            jax.ShapeDtypeStruct((CAND, SD), BF16),
            jax.ShapeDtypeStruct((CAND, 1), F32),
        ],
    )(acts, base, Wa_h, Wpa_t, bias_a, A_a, We_a, Wa_a, strat, wh_col, Wv, hs)


def _select_krn(vrow, vcol, nxt, We_h, cand_o, sel_o, base_o):
    v = vrow[...]
    ii = jax.lax.broadcasted_iota(jnp.int32, (1, CAND), 1)
    rank = jnp.zeros((1, CAND), jnp.int32)
    for t in range(CAND // 256):
        vj = vcol[pl.ds(t * 256, 256), :]
        ji = jax.lax.broadcasted_iota(jnp.int32, (256, 1), 0) + t * 256
        hit = (vj > v) | ((vj == v) & (ji < ii))
        rank = rank + jnp.sum(hit.astype(jnp.int32), axis=0, keepdims=True)
    p = jax.lax.broadcasted_iota(jnp.int32, (TRAJ, 1), 0)
    oh = (rank == p).astype(F32)
    cand_o[...] = _dot(oh, nxt[...]).astype(BF16)
    sel_o[...] = jnp.sum(oh * ii.astype(F32), axis=1, keepdims=True)
    base_o[...] = _dot(cand_o[...], We_h[...])


def _select(vrow, vcol, nxt, We_h):
    return pl.pallas_call(
        _select_krn,
        out_shape=[
            jax.ShapeDtypeStruct((TRAJ, SD), BF16),
            jax.ShapeDtypeStruct((TRAJ, 1), F32),
            jax.ShapeDtypeStruct((TRAJ, SD), F32),
        ],
    )(vrow, vcol, nxt, We_h)


def _head_krn(cand, Wph_t, bias_h, A_h, acts_o):
    logits = _dot(cand[...], Wph_t[...]) + bias_h[...]
    ii = jax.lax.broadcasted_iota(jnp.int32, (TRAJ, PD), 1)
    neg = jnp.float32(-jnp.inf)
    for j in range(8):
        m = jnp.max(logits, axis=1, keepdims=True)
        idx = jnp.min(jnp.where(logits == m, ii, PD), axis=1, keepdims=True)
        sel = ii == idx
        oh = sel.astype(F32)
        acts_o[pl.ds(j * TRAJ, TRAJ), :] = _dot(oh, A_h[...]).astype(BF16)
        logits = jnp.where(sel, neg, logits)


def _head(cand, Wph_t, bias_h, A_h):
    return pl.pallas_call(
        _head_krn,
        out_shape=jax.ShapeDtypeStruct((CAND, AD), BF16),
    )(cand, Wph_t, bias_h, A_h)


def _trace_krn(vrow, sel1, sel0, r0, A_h, out_o):
    v = vrow[...]
    ii = jax.lax.broadcasted_iota(jnp.int32, (1, CAND), 1)
    m = jnp.max(v)
    c2 = jnp.min(jnp.where(v == m, ii, CAND))
    q2 = jnp.remainder(c2, TRAJ)
    i256 = jax.lax.broadcasted_iota(jnp.int32, (1, TRAJ), 1)
    g1 = jnp.sum(jnp.where(i256 == q2, sel1[...], 0.0)).astype(jnp.int32)
    p = jnp.remainder(g1, TRAJ)
    g0 = jnp.sum(jnp.where(i256 == p, sel0[...], 0.0)).astype(jnp.int32)
    ai = jnp.sum(jnp.where(ii == g0, r0[...], 0.0)).astype(jnp.int32)
    ipd = jax.lax.broadcasted_iota(jnp.int32, (1, PD), 1)
    oh = (ipd == ai).astype(F32)
    out_o[...] = jax.lax.dot_general(
        oh, A_h[...], (((1,), (0,)), ((), ())), preferred_element_type=F32,
        precision=jax.lax.Precision.HIGHEST)


def _trace(vrow, sel1, sel0, r0, A_h):
    return pl.pallas_call(
        _trace_krn,
        out_shape=jax.ShapeDtypeStruct((1, AD), F32),
    )(vrow, sel1, sel0, r0, A_h)


def kernel(s_t, strategy0, Wm, Ws, Wp_h, Wp_a, A_h, A_a, We_h, Wa_h, We_a,
           Wa_a, Wv, w_h):
    s_t = s_t.reshape(1, SD)
    bf = lambda x: x.astype(BF16)
    Wph_t, Wph_b = bf(Wp_h[:SD]), bf(Wp_h[SD:])
    Wpa_t, Wpa_b = bf(Wp_a[:SD]), bf(Wp_a[SD:])
    A_h_b, A_a_b = bf(A_h), bf(A_a)
    We_h_b, We_a_b, Wa_h_b, Wa_a_b = bf(We_h), bf(We_a), bf(Wa_h), bf(Wa_a)
    wh_row = w_h.reshape(1, SD)
    wh_col = bf(w_h.reshape(SD, 1))
    Wv_b = bf(Wv)

    strat, bias_h, bias_a, logits0, base0, hs = _prologue(
        s_t, strategy0, bf(Wm), bf(Ws), Wph_t, Wph_b, Wpa_b, We_h_b, wh_row)
    strat_b = bf(strat)

    acts0, r0 = _a0(logits0, logits0.reshape(PD, 1), A_h_b)
    base0r = jnp.broadcast_to(base0, (TRAJ, SD))
    nxt0, vals0 = _expand(7, acts0, base0r, Wa_h_b, Wpa_t, bias_a, A_a_b,
                          We_a_b, Wa_a_b, strat_b, wh_col, Wv_b, hs)

    cand1, sel0, base1 = _select(vals0.reshape(1, CAND), vals0, nxt0, We_h_b)
    acts1 = _head(cand1, Wph_t, bias_h, A_h_b)
    nxt1, vals1 = _expand(6, acts1, base1, Wa_h_b, Wpa_t, bias_a, A_a_b,
                          We_a_b, Wa_a_b, strat_b, wh_col, Wv_b, hs)

    cand2, sel1, base2 = _select(vals1.reshape(1, CAND), vals1, nxt1, We_h_b)
    acts2 = _head(cand2, Wph_t, bias_h, A_h_b)
    _, vals2 = _expand(5, acts2, base2, Wa_h_b, Wpa_t, bias_a, A_a_b,
                       We_a_b, Wa_a_b, strat_b, wh_col, Wv_b, hs)

    out = _trace(vals2.reshape(1, CAND), sel1.reshape(1, TRAJ),
                 sel0.reshape(1, TRAJ), r0.reshape(1, CAND), A_h)
    return out.reshape(AD)
